# baseline (speedup 1.0000x reference)
import functools

import jax
import jax.numpy as jnp
from jax import lax
from jax.experimental import pallas as pl
from jax.experimental.pallas import tpu as pltpu
from jax.experimental.pallas import tpu_sc as plsc

NUM_ENTITIES = 1000000
EMB_DIM = 64
LANES = 16

NUM_CORES = 2
NUM_SUBCORES = 16
NUM_WORKERS = NUM_CORES * NUM_SUBCORES
CHUNK_ROWS = 200
NUM_CHUNKS = NUM_ENTITIES // CHUNK_ROWS

LAST_CHUNK = NUM_CHUNKS - 1
LAST_WORKER = LAST_CHUNK % NUM_WORKERS
LAST_K = LAST_CHUNK // NUM_WORKERS
LAST_K_EVEN = LAST_K % 2 == 0


def _rsqrt_newton(s):
    i = plsc.bitcast(s, jnp.int32)
    i = 0x5F3759DF - (i >> 1)
    y = plsc.bitcast(i, jnp.float32)
    half = 0.5 * s
    y = y * (1.5 - half * y * y)
    y = y * (1.5 - half * y * y)
    y = y * (1.5 - half * y * y)
    return y


def _normalize_rows(x_ref, o_ref):
    def row_body(r, _):
        x0 = x_ref[r, pl.ds(0, LANES)]
        x1 = x_ref[r, pl.ds(LANES, LANES)]
        x2 = x_ref[r, pl.ds(2 * LANES, LANES)]
        x3 = x_ref[r, pl.ds(3 * LANES, LANES)]
        acc = x0 * x0 + x1 * x1 + x2 * x2 + x3 * x3
        lane = lax.iota(jnp.int32, LANES)
        for d in (1, 2, 4, 8):
            perm = jnp.bitwise_xor(lane, d)
            acc = acc + lax.gather(
                acc,
                perm[:, None],
                dimension_numbers=lax.GatherDimensionNumbers(
                    offset_dims=(),
                    collapsed_slice_dims=(0,),
                    start_index_map=(0,),
                ),
                slice_sizes=(1,),
                mode=lax.GatherScatterMode.PROMISE_IN_BOUNDS,
            )
        inv = _rsqrt_newton(acc)
        o_ref[r, pl.ds(0, LANES)] = x0 * inv
        o_ref[r, pl.ds(LANES, LANES)] = x1 * inv
        o_ref[r, pl.ds(2 * LANES, LANES)] = x2 * inv
        o_ref[r, pl.ds(3 * LANES, LANES)] = x3 * inv
        return 0

    lax.fori_loop(0, CHUNK_ROWS, row_body, 0, unroll=4)


def _sc_body(ent_hbm, out_hbm, xa, xb, oa, ob, sem_in, sem_out):
    wid = lax.axis_index("s") * NUM_CORES + lax.axis_index("c")
    n_my = (NUM_CHUNKS - wid + NUM_WORKERS - 1) // NUM_WORKERS

    def chunk_slice(k):
        row0 = pl.multiple_of((wid + k * NUM_WORKERS) * CHUNK_ROWS, 8)
        return pl.ds(row0, CHUNK_ROWS)

    def in_copy(k, buf, sem):
        return pltpu.make_async_copy(ent_hbm.at[chunk_slice(k)], buf, sem)

    def out_copy(k, buf, sem):
        return pltpu.make_async_copy(buf, out_hbm.at[chunk_slice(k)], sem)

    in_copy(0, xa, sem_in).start()

    def loop_body(k, _):
        is_even = lax.rem(k, 2) == 0

        @pl.when(k + 1 < n_my)
        def _():
            @pl.when(is_even)
            def _():
                in_copy(k + 1, xb, sem_in).start()

            @pl.when(jnp.logical_not(is_even))
            def _():
                in_copy(k + 1, xa, sem_in).start()

        def fix_last_row(buf, obuf):
            @pl.when((wid == LAST_WORKER) & (k == LAST_K))
            def _():
                r = CHUNK_ROWS - 1
                for j in range(EMB_DIM // LANES):
                    sl = pl.ds(j * LANES, LANES)
                    obuf[r, sl] = buf[r, sl]

        @pl.when(is_even)
        def _():
            in_copy(k, xa, sem_in).wait()
            _normalize_rows(xa, oa)
            if LAST_K_EVEN:
                fix_last_row(xa, oa)

            @pl.when(k >= 2)
            def _():
                out_copy(k - 2, oa, sem_out).wait()

            out_copy(k, oa, sem_out).start()

        @pl.when(jnp.logical_not(is_even))
        def _():
            in_copy(k, xb, sem_in).wait()
            _normalize_rows(xb, ob)
            if not LAST_K_EVEN:
                fix_last_row(xb, ob)

            @pl.when(k >= 2)
            def _():
                out_copy(k - 2, ob, sem_out).wait()

            out_copy(k, ob, sem_out).start()

        return 0

    lax.fori_loop(0, n_my, loop_body, 0)

    last = n_my - 1

    @pl.when(last >= 1)
    def _():
        @pl.when(lax.rem(last, 2) == 0)
        def _():
            out_copy(last - 1, ob, sem_out).wait()
            out_copy(last, oa, sem_out).wait()

        @pl.when(lax.rem(last, 2) == 1)
        def _():
            out_copy(last - 1, oa, sem_out).wait()
            out_copy(last, ob, sem_out).wait()


_sc_normalize = functools.partial(
    pl.kernel,
    out_type=jax.ShapeDtypeStruct((NUM_ENTITIES, EMB_DIM), jnp.float32),
    mesh=plsc.VectorSubcoreMesh(core_axis_name="c", subcore_axis_name="s"),
    compiler_params=pltpu.CompilerParams(needs_layout_passes=False),
    scratch_types=[
        pltpu.VMEM((CHUNK_ROWS, EMB_DIM), jnp.float32),
        pltpu.VMEM((CHUNK_ROWS, EMB_DIM), jnp.float32),
        pltpu.VMEM((CHUNK_ROWS, EMB_DIM), jnp.float32),
        pltpu.VMEM((CHUNK_ROWS, EMB_DIM), jnp.float32),
        pltpu.SemaphoreType.DMA,
        pltpu.SemaphoreType.DMA,
    ],
)(_sc_body)


def kernel(entity_embds, rel_embds):
    return (_sc_normalize(entity_embds), rel_embds)


# speedup vs baseline: 1.8132x; 1.8132x over previous
import jax
import jax.numpy as jnp
from jax.experimental import pallas as pl

NUM_ENTITIES = 1000000
EMB_DIM = 64
BLOCK_ROWS = 25000


def _normalize_block(ent_ref, out_ref):
    i = pl.program_id(0)
    x = ent_ref[...]
    ss = jnp.sum(x * x, axis=1, keepdims=True)
    inv = jax.lax.rsqrt(ss)
    row = i * BLOCK_ROWS + jax.lax.broadcasted_iota(jnp.int32, (BLOCK_ROWS, 1), 0)
    scale = jnp.where(row == NUM_ENTITIES - 1, 1.0, inv)
    out_ref[...] = x * scale


def kernel(entity_embds, rel_embds):
    grid = NUM_ENTITIES // BLOCK_ROWS
    ent_out = pl.pallas_call(
        _normalize_block,
        grid=(grid,),
        in_specs=[pl.BlockSpec((BLOCK_ROWS, EMB_DIM), lambda i: (i, 0))],
        out_specs=pl.BlockSpec((BLOCK_ROWS, EMB_DIM), lambda i: (i, 0)),
        out_shape=jax.ShapeDtypeStruct((NUM_ENTITIES, EMB_DIM), jnp.float32),
    )(entity_embds)
    return (ent_out, rel_embds)
